# Initial kernel scaffold; baseline (speedup 1.0000x reference)
import jax
import jax.numpy as jnp
from jax.experimental import pallas as pl


def kernel(inputs, edge_index, edge_weight, weight):
    raise NotImplementedError("write your pallas kernel here")


# speedup vs baseline: 3.9748x; 3.9748x over previous
import functools

import jax
import jax.numpy as jnp
from jax import lax
from jax.experimental import pallas as pl
from jax.experimental.pallas import tpu as pltpu
from jax.experimental.pallas import tpu_sc as plsc

V = 10000
C = 128
FOUT = 128
E = 320000
NC = 2
NS = 16
NW = NC * NS
EPT = E // NW
CHUNK = 80
NIT = EPT // CHUNK
LANES = 8

_mesh = plsc.VectorSubcoreMesh(core_axis_name="c", subcore_axis_name="s",
                               num_cores=NC, num_subcores=NS)


@functools.partial(
    pl.kernel,
    out_type=jax.ShapeDtypeStruct((NC * V, C), jnp.float32),
    mesh=_mesh,
    scratch_types=dict(
        accum=pltpu.VMEM_SHARED((V, C), jnp.float32),
        col_v=pltpu.VMEM((CHUNK,), jnp.int32),
        row_v=pltpu.VMEM((CHUNK,), jnp.int32),
        w_v=pltpu.VMEM((CHUNK,), jnp.float32),
        rows_v=pltpu.VMEM((CHUNK, C), jnp.float32),
        zbuf=pltpu.VMEM((125, C), jnp.float32),
        gsem=pltpu.SemaphoreType.DMA,
    ),
)
def _spmv_sc(x_hbm, row_hbm, col_hbm, w_hbm, out_hbm,
             accum, col_v, row_v, w_v, rows_v, zbuf, gsem):
    cid = lax.axis_index("c")
    sid = lax.axis_index("s")
    wid = sid * NC + cid
    base = wid * EPT

    zero16 = jnp.zeros((16,), jnp.float32)

    @pl.loop(0, 125)
    def _zb(j):
        for c in range(LANES):
            zbuf[j, pl.ds(c * 16, 16)] = zero16

    @pl.loop(0, 5)
    def _za(k):
        pltpu.sync_copy(zbuf, accum.at[pl.ds(sid * 625 + k * 125, 125)])

    plsc.subcore_barrier()

    @pl.loop(0, NIT)
    def _edges(i):
        eoff = base + i * CHUNK
        pltpu.sync_copy(col_hbm.at[pl.ds(eoff, CHUNK)], col_v)
        pltpu.sync_copy(row_hbm.at[pl.ds(eoff, CHUNK)], row_v)
        pltpu.sync_copy(w_hbm.at[pl.ds(eoff, CHUNK)], w_v)
        pltpu.async_copy(x_hbm.at[col_v], rows_v, gsem).wait()

        @pl.loop(0, CHUNK // 16)
        def _scale(g):
            wvec = w_v[pl.ds(g * 16, 16)]
            for lane in range(16):
                wv = lax.gather(
                    wvec, jnp.full((16, 1), lane, jnp.int32),
                    lax.GatherDimensionNumbers(offset_dims=(),
                                               collapsed_slice_dims=(0,),
                                               start_index_map=(0,)),
                    slice_sizes=(1,),
                    mode=lax.GatherScatterMode.PROMISE_IN_BOUNDS)
                j = g * 16 + lane
                for c in range(LANES):
                    sl = pl.ds(c * 16, 16)
                    rows_v[j, sl] = rows_v[j, sl] * wv

        pltpu.sync_copy(rows_v, accum.at[row_v], add=True)

    plsc.subcore_barrier()

    pltpu.sync_copy(accum.at[pl.ds(sid * 624, 624)],
                    out_hbm.at[pl.ds(cid * V + sid * 624, 624)])

    @pl.when(sid == NS - 1)
    def _tail():
        pltpu.sync_copy(accum.at[pl.ds(15 * 624, 640)],
                        out_hbm.at[pl.ds(cid * V + 15 * 624, 640)])


_RB = 1000


def _combine_body(a_ref, b_ref, o_ref):
    o_ref[...] = a_ref[...] + b_ref[...]


def _combine(p):
    return pl.pallas_call(
        _combine_body,
        grid=(V // _RB,),
        in_specs=[
            pl.BlockSpec((_RB, C), lambda i: (i, 0)),
            pl.BlockSpec((_RB, C), lambda i: (i + V // _RB, 0)),
        ],
        out_specs=pl.BlockSpec((_RB, C), lambda i: (i, 0)),
        out_shape=jax.ShapeDtypeStruct((V, C), jnp.float32),
    )(p, p)


def _mix_body(x0_ref, x1_ref, p2a_ref, p2b_ref, w_ref, o_ref):
    x0b = x0_ref[...]
    x1b = x1_ref[...]
    x2b = 2.0 * (p2a_ref[...] + p2b_ref[...]) - x0b
    acc = jnp.dot(x0b, w_ref[0], preferred_element_type=jnp.float32)
    acc += jnp.dot(x1b, w_ref[1], preferred_element_type=jnp.float32)
    acc += jnp.dot(x2b, w_ref[2], preferred_element_type=jnp.float32)
    o_ref[...] = acc


def _mix(x0, x1, p2, weight):
    return pl.pallas_call(
        _mix_body,
        grid=(V // _RB,),
        in_specs=[
            pl.BlockSpec((_RB, C), lambda i: (i, 0)),
            pl.BlockSpec((_RB, C), lambda i: (i, 0)),
            pl.BlockSpec((_RB, C), lambda i: (i, 0)),
            pl.BlockSpec((_RB, C), lambda i: (i + V // _RB, 0)),
            pl.BlockSpec((3, C, FOUT), lambda i: (0, 0, 0)),
        ],
        out_specs=pl.BlockSpec((_RB, FOUT), lambda i: (i, 0)),
        out_shape=jax.ShapeDtypeStruct((V, FOUT), jnp.float32),
    )(x0, x1, p2, p2, weight)


def kernel(inputs, edge_index, edge_weight, weight):
    B, Fin, V_, X, Y, Z = inputs.shape
    K, _, Fout = weight.shape
    x0 = inputs.reshape(Fin, V_).T
    row = edge_index[0]
    col = edge_index[1]
    p1 = _spmv_sc(x0, row, col, edge_weight)
    x1 = _combine(p1)
    p2 = _spmv_sc(x1, row, col, edge_weight)
    y = _mix(x0, x1, p2, weight)
    return y.T.reshape(B, Fout, V_, X, Y, Z)


# speedup vs baseline: 5.8606x; 1.4745x over previous
import functools

import jax
import jax.numpy as jnp
from jax import lax
from jax.experimental import pallas as pl
from jax.experimental.pallas import tpu as pltpu
from jax.experimental.pallas import tpu_sc as plsc

V = 10000
C = 128
FOUT = 128
E = 320000
NC = 2
NS = 16
NW = NC * NS
EPT = E // NW
CHUNK = 128
NPC = -(-EPT // CHUNK)
BLK = (NPC + 1) // 2
LANES = 8

_mesh = plsc.VectorSubcoreMesh(core_axis_name="c", subcore_axis_name="s",
                               num_cores=NC, num_subcores=NS)


@functools.partial(
    pl.kernel,
    out_type=jax.ShapeDtypeStruct((NC * V, C), jnp.float32),
    mesh=_mesh,
    scratch_types=dict(
        accum=pltpu.VMEM_SHARED((V, C), jnp.float32),
        col_v=pltpu.VMEM((BLK, CHUNK), jnp.int32),
        row_v=pltpu.VMEM((BLK, CHUNK), jnp.int32),
        w_v=pltpu.VMEM((BLK, CHUNK), jnp.float32),
        rows_a=pltpu.VMEM((CHUNK, C), jnp.float32),
        rows_b=pltpu.VMEM((CHUNK, C), jnp.float32),
        ga=pltpu.SemaphoreType.DMA,
        gb=pltpu.SemaphoreType.DMA,
        sa=pltpu.SemaphoreType.DMA,
        sb=pltpu.SemaphoreType.DMA,
    ),
)
def _spmv_sc(x_hbm, row_hbm, col_hbm, w_hbm, out_hbm,
             accum, col_v, row_v, w_v, rows_a, rows_b, ga, gb, sa, sb):
    cid = lax.axis_index("c")
    sid = lax.axis_index("s")
    wid = sid * NC + cid

    rows = (rows_a, rows_b)
    gsem = (ga, gb)
    ssem = (sa, sb)

    zero16 = jnp.zeros((16,), jnp.float32)

    @pl.loop(0, CHUNK)
    def _zb(j):
        for c in range(LANES):
            rows_a[j, pl.ds(c * 16, 16)] = zero16

    @pl.loop(0, 4)
    def _za(k):
        pltpu.sync_copy(rows_a, accum.at[pl.ds(sid * 625 + k * CHUNK, CHUNK)])

    pltpu.sync_copy(rows_a.at[pl.ds(0, 113)],
                    accum.at[pl.ds(sid * 625 + 512, 113)])

    plsc.subcore_barrier()

    def g_start(it, b):
        pltpu.async_copy(x_hbm.at[col_v.at[it]], rows[b], gsem[b])

    def g_wait(b):
        pltpu.make_async_copy(x_hbm.at[col_v.at[0]], rows[b], gsem[b]).wait()

    def s_start(it, b):
        pltpu.async_copy(rows[b], accum.at[row_v.at[it]], ssem[b], add=True)

    def s_wait(b):
        pltpu.make_async_copy(rows[b], accum.at[row_v.at[0]], ssem[b]).wait()

    def scale(it, b):
        rbuf = rows[b]

        @pl.loop(0, CHUNK // 16)
        def _scale(g):
            wvec = w_v[it, pl.ds(g * 16, 16)]
            for lane in range(16):
                wv = lax.gather(
                    wvec, jnp.full((16, 1), lane, jnp.int32),
                    lax.GatherDimensionNumbers(offset_dims=(),
                                               collapsed_slice_dims=(0,),
                                               start_index_map=(0,)),
                    slice_sizes=(1,),
                    mode=lax.GatherScatterMode.PROMISE_IN_BOUNDS)
                j = g * 16 + lane
                for c in range(LANES):
                    sl = pl.ds(c * 16, 16)
                    rbuf[j, sl] = rbuf[j, sl] * wv

    def run_block(n):
        def handle(it, b):
            o = 1 - b
            s_wait(o)

            @pl.when(it + 1 < n)
            def _pref():
                g_start(it + 1, o)

            g_wait(b)
            scale(it, b)
            s_start(it, b)

        g_start(0, 0)
        g_wait(0)
        g_start(1, 1)
        scale(0, 0)
        s_start(0, 0)

        @pl.loop(0, (n - 1) // 2)
        def _pipe(i):
            handle(1 + 2 * i, 1)
            handle(2 + 2 * i, 0)

        if (n - 1) % 2 == 1:
            handle(n - 1, 1)
        s_wait((n - 1) % 2)

    for c0, n in ((0, BLK), (BLK, NPC - BLK)):
        pltpu.sync_copy(col_hbm.at[wid, pl.ds(c0, n)], col_v.at[pl.ds(0, n)])
        pltpu.sync_copy(row_hbm.at[wid, pl.ds(c0, n)], row_v.at[pl.ds(0, n)])
        pltpu.sync_copy(w_hbm.at[wid, pl.ds(c0, n)], w_v.at[pl.ds(0, n)])
        run_block(n)

    plsc.subcore_barrier()

    pltpu.sync_copy(accum.at[pl.ds(sid * 624, 624)],
                    out_hbm.at[pl.ds(cid * V + sid * 624, 624)])

    @pl.when(sid == NS - 1)
    def _tail():
        pltpu.sync_copy(accum.at[pl.ds(15 * 624, 640)],
                        out_hbm.at[pl.ds(cid * V + 15 * 624, 640)])


_RB = 1000


def _combine_body(a_ref, b_ref, o_ref):
    o_ref[...] = a_ref[...] + b_ref[...]


def _combine(p):
    return pl.pallas_call(
        _combine_body,
        grid=(V // _RB,),
        in_specs=[
            pl.BlockSpec((_RB, C), lambda i: (i, 0)),
            pl.BlockSpec((_RB, C), lambda i: (i + V // _RB, 0)),
        ],
        out_specs=pl.BlockSpec((_RB, C), lambda i: (i, 0)),
        out_shape=jax.ShapeDtypeStruct((V, C), jnp.float32),
    )(p, p)


def _mix_body(x0_ref, x1_ref, p2a_ref, p2b_ref, w_ref, o_ref):
    x0b = x0_ref[...]
    x1b = x1_ref[...]
    x2b = 2.0 * (p2a_ref[...] + p2b_ref[...]) - x0b
    acc = jnp.dot(x0b, w_ref[0], preferred_element_type=jnp.float32)
    acc += jnp.dot(x1b, w_ref[1], preferred_element_type=jnp.float32)
    acc += jnp.dot(x2b, w_ref[2], preferred_element_type=jnp.float32)
    o_ref[...] = acc


def _mix(x0, x1, p2, weight):
    return pl.pallas_call(
        _mix_body,
        grid=(V // _RB,),
        in_specs=[
            pl.BlockSpec((_RB, C), lambda i: (i, 0)),
            pl.BlockSpec((_RB, C), lambda i: (i, 0)),
            pl.BlockSpec((_RB, C), lambda i: (i, 0)),
            pl.BlockSpec((_RB, C), lambda i: (i + V // _RB, 0)),
            pl.BlockSpec((3, C, FOUT), lambda i: (0, 0, 0)),
        ],
        out_specs=pl.BlockSpec((_RB, FOUT), lambda i: (i, 0)),
        out_shape=jax.ShapeDtypeStruct((V, FOUT), jnp.float32),
    )(x0, x1, p2, p2, weight)


def _pad_edges(a, fill):
    per = a.reshape(NW, EPT)
    pad = jnp.full((NW, NPC * CHUNK - EPT), fill, a.dtype)
    return jnp.concatenate([per, pad], axis=1).reshape(NW, NPC, CHUNK)


def kernel(inputs, edge_index, edge_weight, weight):
    B, Fin, V_, X, Y, Z = inputs.shape
    K, _, Fout = weight.shape
    x0 = inputs.reshape(Fin, V_).T
    row = _pad_edges(edge_index[0], 0)
    col = _pad_edges(edge_index[1], 0)
    w3 = _pad_edges(edge_weight, 0.0)
    p1 = _spmv_sc(x0, row, col, w3)
    x1 = _combine(p1)
    p2 = _spmv_sc(x1, row, col, w3)
    y = _mix(x0, x1, p2, weight)
    return y.T.reshape(B, Fout, V_, X, Y, Z)
